# baseline (speedup 1.0000x reference)
import functools

import jax
import jax.numpy as jnp
from jax import lax
from jax.experimental import pallas as pl
from jax.experimental.pallas import tpu as pltpu
from jax.experimental.pallas import tpu_sc as plsc

N = 10000
D = 128
E = 320000
NC = 2
NS = 16
NW = NC * NS
PER_W = E // NW
CH = 64
NCH = PER_W // CH
REM = PER_W - NCH * CH
NBUF = 6
STG = 4
SIDEPTH = 6
STG2 = 5
UNROLL = 6
N_ACC = N
ZSTRIPE = 624
ZFULL = ZSTRIPE - ZSTRIPE % CH
ZTAIL = ZSTRIPE - ZFULL
OUT_TAIL = N - NS * ZSTRIPE


def _sc_aggregate(x, adj):
    mesh = plsc.VectorSubcoreMesh(core_axis_name="c", subcore_axis_name="s")

    @functools.partial(
        pl.kernel,
        out_type=jax.ShapeDtypeStruct((NC, N, D), jnp.float32),
        mesh=mesh,
        scratch_types=(
            [pltpu.VMEM((CH,), jnp.int32)] * SIDEPTH
            + [pltpu.VMEM((CH,), jnp.int32)] * NBUF
            + [pltpu.VMEM((CH, D), jnp.float32)] * NBUF
            + [pltpu.VMEM_SHARED((N_ACC, D), jnp.float32)]
            + [pltpu.SemaphoreType.DMA] * (SIDEPTH + 3 * NBUF)
        ),
    )
    def k(x_hbm, adj_hbm, out_hbm, *refs):
        sring = refs[0:SIDEPTH]
        didx = refs[SIDEPTH:SIDEPTH + NBUF]
        rows = refs[SIDEPTH + NBUF:SIDEPTH + 2 * NBUF]
        acc = refs[SIDEPTH + 2 * NBUF]
        sems = refs[SIDEPTH + 2 * NBUF + 1:]
        sem_sr = sems[0:SIDEPTH]
        sem_d = sems[SIDEPTH:SIDEPTH + NBUF]
        sem_g = sems[SIDEPTH + NBUF:SIDEPTH + 2 * NBUF]
        sem_s = sems[SIDEPTH + 2 * NBUF:SIDEPTH + 3 * NBUF]

        cid = lax.axis_index("c")
        sid = lax.axis_index("s")
        wid = sid * NC + cid
        base = wid * PER_W

        cp_sr = [pltpu.async_copy(adj_hbm.at[pl.ds(base + s * CH, CH)],
                                  sring[s], sem_sr[s]) for s in range(SIDEPTH)]
        cp_d = [pltpu.async_copy(adj_hbm.at[pl.ds(E + base + b * CH, CH)],
                                 didx[b], sem_d[b]) for b in range(STG)]

        zero16 = jnp.zeros((16,), jnp.float32)

        @pl.loop(0, CH)
        def _(r):
            @pl.loop(0, D, step=16)
            def _(c2):
                rows[0][r, pl.ds(c2, 16)] = zero16

        @pl.loop(0, ZFULL, step=CH)
        def _(i):
            pltpu.sync_copy(rows[0], acc.at[pl.ds(sid * ZSTRIPE + i, CH)])

        pltpu.sync_copy(rows[0].at[pl.ds(0, ZTAIL)],
                        acc.at[pl.ds(sid * ZSTRIPE + ZFULL, ZTAIL)])

        @pl.when(sid == 0)
        def _():
            pltpu.sync_copy(rows[0].at[pl.ds(0, OUT_TAIL)],
                            acc.at[pl.ds(NS * ZSTRIPE, OUT_TAIL)])

        cp_g = []
        for b in range(STG):
            cp_sr[b].wait()
            cp_g.append(pltpu.async_copy(
                x_hbm.at[sring[b].at[...]], rows[b], sem_g[b]))

        all_cp_sr = cp_sr
        all_cp_d = cp_d + [pltpu.make_async_copy(
            adj_hbm.at[pl.ds(E + base + b * CH, CH)], didx[b], sem_d[b])
            for b in range(STG, NBUF)]
        all_cp_g = cp_g + [pltpu.make_async_copy(
            x_hbm.at[sring[b % SIDEPTH].at[...]], rows[b], sem_g[b])
            for b in range(STG, NBUF)]
        cp_sc = [pltpu.make_async_copy(rows[b], acc.at[didx[b]], sem_s[b])
                 for b in range(NBUF)]

        plsc.subcore_barrier()

        @pl.loop(0, NCH, step=UNROLL)
        def _(i):
            for b2 in range(UNROLL):
                j = i + b2
                b = b2 % NBUF
                pb = (b2 + STG) % NBUF
                ps = (b2 + STG) % SIDEPTH
                ps2 = (b2 + STG2) % SIDEPTH

                @pl.when(j >= NBUF - STG)
                def _():
                    cp_sc[pb].wait()

                @pl.when(j + STG < NCH)
                def _():
                    pltpu.async_copy(
                        adj_hbm.at[pl.ds(E + base + (j + STG) * CH, CH)],
                        didx[pb], sem_d[pb])
                    all_cp_sr[ps].wait()
                    pltpu.async_copy(
                        x_hbm.at[sring[ps].at[...]], rows[pb], sem_g[pb])

                @pl.when(jnp.logical_and(j >= SIDEPTH - STG2,
                                         j + STG2 < NCH))
                def _():
                    pltpu.async_copy(
                        adj_hbm.at[pl.ds(base + (j + STG2) * CH, CH)],
                        sring[ps2], sem_sr[ps2])

                all_cp_g[b].wait()
                all_cp_d[b].wait()
                pltpu.async_copy(rows[b], acc.at[didx[b]], sem_s[b], add=True)

        for t in range(NBUF - STG):
            cp_sc[(NCH - 1 - t) % NBUF].wait()
        pltpu.sync_copy(adj_hbm.at[pl.ds(base + NCH * CH, REM)],
                        sring[0].at[pl.ds(0, REM)])
        pltpu.sync_copy(adj_hbm.at[pl.ds(E + base + NCH * CH, REM)],
                        didx[0].at[pl.ds(0, REM)])
        pltpu.sync_copy(x_hbm.at[sring[0].at[pl.ds(0, REM)]],
                        rows[0].at[pl.ds(0, REM)])
        pltpu.sync_copy(rows[0].at[pl.ds(0, REM)],
                        acc.at[didx[0].at[pl.ds(0, REM)]], add=True)

        plsc.subcore_barrier()

        pltpu.sync_copy(acc.at[pl.ds(sid * ZSTRIPE, ZSTRIPE)],
                        out_hbm.at[cid].at[pl.ds(sid * ZSTRIPE, ZSTRIPE)])

        @pl.when(sid == 0)
        def _():
            pltpu.sync_copy(acc.at[pl.ds(NS * ZSTRIPE, OUT_TAIL)],
                            out_hbm.at[cid].at[pl.ds(NS * ZSTRIPE, OUT_TAIL)])

    return k(x, adj.reshape(-1))


BLK = 2000


def _tc_mlp(x, agg, W1, b1r, W2, b2r, eps_row):
    def body(x_ref, a_ref, w1_ref, b1_ref, w2_ref, b2_ref, e_ref, o_ref):
        h = (1.0 + e_ref[...]) * x_ref[...] + a_ref[0] + a_ref[1]
        h = jnp.dot(h, w1_ref[...], preferred_element_type=jnp.float32)
        h = jnp.maximum(h + b1_ref[...], 0.0)
        o = jnp.dot(h, w2_ref[...], preferred_element_type=jnp.float32)
        o_ref[...] = o + b2_ref[...]

    return pl.pallas_call(
        body,
        grid=(N // BLK,),
        in_specs=[
            pl.BlockSpec((BLK, D), lambda i: (i, 0)),
            pl.BlockSpec((NC, BLK, D), lambda i: (0, i, 0)),
            pl.BlockSpec((D, D), lambda i: (0, 0)),
            pl.BlockSpec((1, D), lambda i: (0, 0)),
            pl.BlockSpec((D, D), lambda i: (0, 0)),
            pl.BlockSpec((1, D), lambda i: (0, 0)),
            pl.BlockSpec((1, D), lambda i: (0, 0)),
        ],
        out_specs=pl.BlockSpec((BLK, D), lambda i: (i, 0)),
        out_shape=jax.ShapeDtypeStruct((N, D), jnp.float32),
    )(x, agg, W1, b1r, W2, b2r, eps_row)


def kernel(x, adj, W1, b1, W2, b2, eps):
    adj32 = adj.astype(jnp.int32)
    agg = _sc_aggregate(x, adj32)
    b1r = b1.reshape(1, D)
    b2r = b2.reshape(1, D)
    eps_row = jnp.broadcast_to(eps, (1, D)).astype(jnp.float32)
    return _tc_mlp(x, agg, W1, b1r, W2, b2r, eps_row)


# speedup vs baseline: 1.0260x; 1.0260x over previous
import functools

import jax
import jax.numpy as jnp
from jax import lax
from jax.experimental import pallas as pl
from jax.experimental.pallas import tpu as pltpu
from jax.experimental.pallas import tpu_sc as plsc

N = 10000
D = 128
E = 320000
NC = 2
NS = 16
NW = NC * NS
PER_W = E // NW
CH = 128
NCH = PER_W // CH
REM = PER_W - NCH * CH
NBUF = 3
STG = 2
SIDEPTH = 6
STG2 = 4
UNROLL = 6
N_ACC = N
ZSTRIPE = 624
ZFULL = ZSTRIPE - ZSTRIPE % CH
ZTAIL = ZSTRIPE - ZFULL
OUT_TAIL = N - NS * ZSTRIPE


def _sc_aggregate(x, adj):
    mesh = plsc.VectorSubcoreMesh(core_axis_name="c", subcore_axis_name="s")

    @functools.partial(
        pl.kernel,
        out_type=jax.ShapeDtypeStruct((NC, N, D), jnp.float32),
        mesh=mesh,
        scratch_types=(
            [pltpu.VMEM((CH,), jnp.int32)] * SIDEPTH
            + [pltpu.VMEM((CH,), jnp.int32)] * NBUF
            + [pltpu.VMEM((CH, D), jnp.float32)] * NBUF
            + [pltpu.VMEM_SHARED((N_ACC, D), jnp.float32)]
            + [pltpu.SemaphoreType.DMA] * (SIDEPTH + 3 * NBUF)
        ),
    )
    def k(x_hbm, adj_hbm, out_hbm, *refs):
        sring = refs[0:SIDEPTH]
        didx = refs[SIDEPTH:SIDEPTH + NBUF]
        rows = refs[SIDEPTH + NBUF:SIDEPTH + 2 * NBUF]
        acc = refs[SIDEPTH + 2 * NBUF]
        sems = refs[SIDEPTH + 2 * NBUF + 1:]
        sem_sr = sems[0:SIDEPTH]
        sem_d = sems[SIDEPTH:SIDEPTH + NBUF]
        sem_g = sems[SIDEPTH + NBUF:SIDEPTH + 2 * NBUF]
        sem_s = sems[SIDEPTH + 2 * NBUF:SIDEPTH + 3 * NBUF]

        cid = lax.axis_index("c")
        sid = lax.axis_index("s")
        wid = sid * NC + cid
        base = wid * PER_W

        cp_sr = [pltpu.async_copy(adj_hbm.at[pl.ds(base + s * CH, CH)],
                                  sring[s], sem_sr[s]) for s in range(SIDEPTH)]
        cp_d = [pltpu.async_copy(adj_hbm.at[pl.ds(E + base + b * CH, CH)],
                                 didx[b], sem_d[b]) for b in range(STG)]

        zero16 = jnp.zeros((16,), jnp.float32)

        @pl.loop(0, CH)
        def _(r):
            @pl.loop(0, D, step=16)
            def _(c2):
                rows[0][r, pl.ds(c2, 16)] = zero16

        @pl.loop(0, ZFULL, step=CH)
        def _(i):
            pltpu.sync_copy(rows[0], acc.at[pl.ds(sid * ZSTRIPE + i, CH)])

        pltpu.sync_copy(rows[0].at[pl.ds(0, ZTAIL)],
                        acc.at[pl.ds(sid * ZSTRIPE + ZFULL, ZTAIL)])

        @pl.when(sid == 0)
        def _():
            pltpu.sync_copy(rows[0].at[pl.ds(0, OUT_TAIL)],
                            acc.at[pl.ds(NS * ZSTRIPE, OUT_TAIL)])

        cp_g = []
        for b in range(STG):
            cp_sr[b].wait()
            cp_g.append(pltpu.async_copy(
                x_hbm.at[sring[b].at[...]], rows[b], sem_g[b]))

        all_cp_sr = cp_sr
        all_cp_d = cp_d + [pltpu.make_async_copy(
            adj_hbm.at[pl.ds(E + base + b * CH, CH)], didx[b], sem_d[b])
            for b in range(STG, NBUF)]
        all_cp_g = cp_g + [pltpu.make_async_copy(
            x_hbm.at[sring[b % SIDEPTH].at[...]], rows[b], sem_g[b])
            for b in range(STG, NBUF)]
        cp_sc = [pltpu.make_async_copy(rows[b], acc.at[didx[b]], sem_s[b])
                 for b in range(NBUF)]

        plsc.subcore_barrier()

        @pl.loop(0, NCH, step=UNROLL)
        def _(i):
            for b2 in range(UNROLL):
                j = i + b2
                b = b2 % NBUF
                pb = (b2 + STG) % NBUF
                ps = (b2 + STG) % SIDEPTH
                ps2 = (b2 + STG2) % SIDEPTH

                @pl.when(j >= NBUF - STG)
                def _():
                    cp_sc[pb].wait()

                @pl.when(j + STG < NCH)
                def _():
                    pltpu.async_copy(
                        adj_hbm.at[pl.ds(E + base + (j + STG) * CH, CH)],
                        didx[pb], sem_d[pb])
                    all_cp_sr[ps].wait()
                    pltpu.async_copy(
                        x_hbm.at[sring[ps].at[...]], rows[pb], sem_g[pb])

                @pl.when(jnp.logical_and(j >= SIDEPTH - STG2,
                                         j + STG2 < NCH))
                def _():
                    pltpu.async_copy(
                        adj_hbm.at[pl.ds(base + (j + STG2) * CH, CH)],
                        sring[ps2], sem_sr[ps2])

                all_cp_g[b].wait()
                all_cp_d[b].wait()
                pltpu.async_copy(rows[b], acc.at[didx[b]], sem_s[b], add=True)

        for t in range(NBUF - STG):
            cp_sc[(NCH - 1 - t) % NBUF].wait()
        pltpu.sync_copy(adj_hbm.at[pl.ds(base + NCH * CH, REM)],
                        sring[0].at[pl.ds(0, REM)])
        pltpu.sync_copy(adj_hbm.at[pl.ds(E + base + NCH * CH, REM)],
                        didx[0].at[pl.ds(0, REM)])
        pltpu.sync_copy(x_hbm.at[sring[0].at[pl.ds(0, REM)]],
                        rows[0].at[pl.ds(0, REM)])
        pltpu.sync_copy(rows[0].at[pl.ds(0, REM)],
                        acc.at[didx[0].at[pl.ds(0, REM)]], add=True)

        plsc.subcore_barrier()

        pltpu.sync_copy(acc.at[pl.ds(sid * ZSTRIPE, ZSTRIPE)],
                        out_hbm.at[cid].at[pl.ds(sid * ZSTRIPE, ZSTRIPE)])

        @pl.when(sid == 0)
        def _():
            pltpu.sync_copy(acc.at[pl.ds(NS * ZSTRIPE, OUT_TAIL)],
                            out_hbm.at[cid].at[pl.ds(NS * ZSTRIPE, OUT_TAIL)])

    return k(x, adj.reshape(-1))


BLK = 2000


def _tc_mlp(x, agg, W1, b1r, W2, b2r, eps_row):
    def body(x_ref, a_ref, w1_ref, b1_ref, w2_ref, b2_ref, e_ref, o_ref):
        h = (1.0 + e_ref[...]) * x_ref[...] + a_ref[0] + a_ref[1]
        h = jnp.dot(h, w1_ref[...], preferred_element_type=jnp.float32)
        h = jnp.maximum(h + b1_ref[...], 0.0)
        o = jnp.dot(h, w2_ref[...], preferred_element_type=jnp.float32)
        o_ref[...] = o + b2_ref[...]

    return pl.pallas_call(
        body,
        grid=(N // BLK,),
        in_specs=[
            pl.BlockSpec((BLK, D), lambda i: (i, 0)),
            pl.BlockSpec((NC, BLK, D), lambda i: (0, i, 0)),
            pl.BlockSpec((D, D), lambda i: (0, 0)),
            pl.BlockSpec((1, D), lambda i: (0, 0)),
            pl.BlockSpec((D, D), lambda i: (0, 0)),
            pl.BlockSpec((1, D), lambda i: (0, 0)),
            pl.BlockSpec((1, D), lambda i: (0, 0)),
        ],
        out_specs=pl.BlockSpec((BLK, D), lambda i: (i, 0)),
        out_shape=jax.ShapeDtypeStruct((N, D), jnp.float32),
    )(x, agg, W1, b1r, W2, b2r, eps_row)


def kernel(x, adj, W1, b1, W2, b2, eps):
    adj32 = adj.astype(jnp.int32)
    agg = _sc_aggregate(x, adj32)
    b1r = b1.reshape(1, D)
    b2r = b2.reshape(1, D)
    eps_row = jnp.broadcast_to(eps, (1, D)).astype(jnp.float32)
    return _tc_mlp(x, agg, W1, b1r, W2, b2r, eps_row)
